# Initial kernel scaffold; baseline (speedup 1.0000x reference)
import jax
import jax.numpy as jnp
from jax.experimental import pallas as pl


def kernel(x, W, b):
    raise NotImplementedError("write your pallas kernel here")


# speedup vs baseline: 1.6209x; 1.6209x over previous
import functools

import jax
import jax.numpy as jnp
from jax.experimental import pallas as pl
from jax.experimental.pallas import tpu as pltpu

TOKENS_PER_BLOCK = 1024


def _router_block(x_ref, w_ref, b_ref, idx_ref, val_ref):
    x_blk = x_ref[...]
    w = w_ref[...]
    logits = jax.lax.dot_general(
        x_blk, w, (((1,), (1,)), ((), ())),
        preferred_element_type=jnp.float32,
    ) + b_ref[...]

    bt, ne = logits.shape
    lane = jax.lax.broadcasted_iota(jnp.int32, (bt, ne), 1)

    m1 = jnp.max(logits, axis=1, keepdims=True)
    i1 = jnp.min(jnp.where(logits == m1, lane, ne), axis=1, keepdims=True)
    masked = jnp.where(lane == i1, -jnp.inf, logits)
    m2 = jnp.max(masked, axis=1, keepdims=True)
    i2 = jnp.min(jnp.where(masked == m2, lane, ne), axis=1, keepdims=True)

    denom = jnp.sum(jnp.exp(logits - m1), axis=1, keepdims=True)
    v1 = 1.0 / denom
    v2 = jnp.exp(m2 - m1) * v1

    idx_ref[...] = jnp.concatenate([i1, i2], axis=1)
    val_ref[...] = jnp.concatenate([v1, v2], axis=1)


@jax.jit
def kernel(x, W, b):
    tokens, d = x.shape
    ne = W.shape[0]
    bt = TOKENS_PER_BLOCK
    grid = (tokens // bt,)
    idx, vals = pl.pallas_call(
        _router_block,
        grid=grid,
        in_specs=[
            pl.BlockSpec((bt, d), lambda i: (i, 0)),
            pl.BlockSpec((ne, d), lambda i: (0, 0)),
            pl.BlockSpec((1, ne), lambda i: (0, 0)),
        ],
        out_specs=[
            pl.BlockSpec((bt, 2), lambda i: (i, 0)),
            pl.BlockSpec((bt, 2), lambda i: (i, 0)),
        ],
        out_shape=[
            jax.ShapeDtypeStruct((tokens, 2), jnp.int32),
            jax.ShapeDtypeStruct((tokens, 2), jnp.float32),
        ],
    )(x, W, b.reshape(1, ne))
    return idx, vals
